# Initial kernel scaffold; baseline (speedup 1.0000x reference)
import jax
import jax.numpy as jnp
from jax.experimental import pallas as pl


def kernel(x, edge_index, W1, b1, W2, b2, W3, b3):
    raise NotImplementedError("write your pallas kernel here")


# speedup vs baseline: 15.9916x; 15.9916x over previous
import functools

import jax
import jax.numpy as jnp
from jax import lax
from jax.experimental import pallas as pl
from jax.experimental.pallas import tpu as pltpu
from jax.experimental.pallas import tpu_sc as plsc

NC = 2
NS = 16
NW = NC * NS
C = 128
NB = 4


def _agg_body(CH, RPT, g_hbm, src3d, dst3d, z_hbm, out_hbm,
              acc, src_v, dst_v, r0, r1, r2, r3, s0, s1, s2, s3):
  rows = (r0, r1, r2, r3)
  sems = (s0, s1, s2, s3)
  cid = lax.axis_index("c")
  sid = lax.axis_index("s")
  wid = sid * NC + cid

  pltpu.sync_copy(src3d.at[wid], src_v)
  pltpu.sync_copy(dst3d.at[wid], dst_v)
  pltpu.sync_copy(z_hbm.at[pl.ds(sid * RPT, RPT)],
                  acc.at[pl.ds(sid * RPT, RPT)])
  plsc.subcore_barrier()

  for b in range(NB):
    pltpu.async_copy(g_hbm.at[src_v.at[b]], rows[b], sems[b])

  def step(c0, carry):
    for b in range(NB):
      c = c0 * NB + b
      pltpu.make_async_copy(g_hbm.at[src_v.at[c]], rows[b], sems[b]).wait()
      pltpu.sync_copy(rows[b], acc.at[dst_v.at[c]], add=True)

      @pl.when(c + NB < CH)
      def _():
        pltpu.async_copy(g_hbm.at[src_v.at[c + NB]], rows[b], sems[b])
    return carry

  lax.fori_loop(0, CH // NB, step, 0)
  plsc.subcore_barrier()
  pltpu.sync_copy(acc.at[pl.ds(sid * RPT, RPT)],
                  out_hbm.at[cid, pl.ds(sid * RPT, RPT)])


@functools.cache
def _make_agg(n_table, d, ch, nrows):
  rpt = nrows // NS
  mesh = plsc.VectorSubcoreMesh(core_axis_name="c", subcore_axis_name="s",
                                num_cores=NC, num_subcores=NS)
  return pl.kernel(
      functools.partial(_agg_body, ch, rpt),
      out_type=jax.ShapeDtypeStruct((NC, nrows, d), jnp.float32),
      mesh=mesh,
      compiler_params=pltpu.CompilerParams(use_tc_tiling_on_sc=False),
      scratch_types=[
          pltpu.VMEM_SHARED((nrows, d), jnp.float32),
          pltpu.VMEM((ch, C), jnp.int32),
          pltpu.VMEM((ch, C), jnp.int32),
          *[pltpu.VMEM((C, d), jnp.float32) for _ in range(NB)],
          *[pltpu.SemaphoreType.DMA for _ in range(NB)],
      ],
  )


def _tc_head(dp_ref, x_ref, w_ref, g_ref, dinv_ref):
  deg = dp_ref[:, 0:1] + dp_ref[:, 1:2] + 1.0
  dinv = lax.rsqrt(jnp.maximum(deg, 1.0))
  g_ref[...] = dinv * jnp.dot(x_ref[...], w_ref[...],
                              preferred_element_type=jnp.float32)
  dinv_ref[...] = dinv


def _tc_mid(ap_ref, g_ref, dinv_ref, b_ref, w_ref, gout_ref):
  dinv = dinv_ref[...]
  t = dinv * (ap_ref[0] + ap_ref[1] + g_ref[...]) + b_ref[...]
  t = jnp.maximum(t, 0.0)
  gout_ref[...] = dinv * jnp.dot(t, w_ref[...],
                                 preferred_element_type=jnp.float32)


def _tc_tail(ap_ref, g_ref, dinv_ref, b_ref, out_ref):
  out_ref[...] = (dinv_ref[...] * (ap_ref[0] + ap_ref[1] + g_ref[...])
                  + b_ref[...])


def kernel(x, edge_index, W1, b1, W2, b2, W3, b3):
  n, d_in = x.shape
  e = edge_index.shape[1]
  dh = W1.shape[1]
  do = W3.shape[1]
  d3 = 8

  ch = -(-e // (NW * C))
  ch = -(-ch // NB) * NB
  pade = NW * ch * C
  nrows = -(-(n + 1) // (NS * 8)) * (NS * 8)

  src = edge_index[0]
  dst = edge_index[1]
  npad = pade - e
  srcp = jnp.concatenate([src, jnp.zeros((npad,), src.dtype)])
  dstp = jnp.concatenate([dst, jnp.full((npad,), n, dst.dtype)])
  src3d = srcp.reshape(NW, ch, C)
  dst3d = dstp.reshape(NW, ch, C)

  z64 = jnp.zeros((nrows, dh), jnp.float32)
  z8 = jnp.zeros((nrows, d3), jnp.float32)
  ones8 = jnp.ones((n, d3), jnp.float32)
  W3p = jnp.concatenate([W3, jnp.zeros((dh, d3 - do), W3.dtype)], axis=1)
  b3p = jnp.concatenate([b3, jnp.zeros((d3 - do,), b3.dtype)])

  agg64 = _make_agg(n, dh, ch, nrows)
  agg8 = _make_agg(n, d3, ch, nrows)

  degp = agg8(ones8, src3d, dst3d, z8)
  dp = jnp.transpose(degp[:, :n, 0])

  g1, dinv = pl.pallas_call(
      _tc_head,
      out_shape=(jax.ShapeDtypeStruct((n, dh), jnp.float32),
                 jax.ShapeDtypeStruct((n, 1), jnp.float32)),
  )(dp, x, W1)

  a1 = agg64(g1, src3d, dst3d, z64)
  g2 = pl.pallas_call(
      _tc_mid, out_shape=jax.ShapeDtypeStruct((n, dh), jnp.float32),
  )(a1[:, :n, :], g1, dinv, b1.reshape(1, dh), W2)

  a2 = agg64(g2, src3d, dst3d, z64)
  g3 = pl.pallas_call(
      _tc_mid, out_shape=jax.ShapeDtypeStruct((n, d3), jnp.float32),
  )(a2[:, :n, :], g2, dinv, b2.reshape(1, dh), W3p)

  a3 = agg8(g3, src3d, dst3d, z8)
  dxp = pl.pallas_call(
      _tc_tail, out_shape=jax.ShapeDtypeStruct((n, d3), jnp.float32),
  )(a3[:, :n, :], g3, dinv, b3p.reshape(1, d3))

  return dxp[:, :do]


# speedup vs baseline: 20.4159x; 1.2767x over previous
import functools

import jax
import jax.numpy as jnp
from jax import lax
from jax.experimental import pallas as pl
from jax.experimental.pallas import tpu as pltpu
from jax.experimental.pallas import tpu_sc as plsc

NC = 2
NS = 16
NW = NC * NS
C = 128
NB = 8
LA = 4


def _agg_body(CH, RPT, g_hbm, src3d, dst3d, z_hbm, out_hbm, acc,
              src_v, dst_v, *bufs):
  rows = bufs[:NB]
  gsems = bufs[NB:2 * NB]
  ssems = bufs[2 * NB:3 * NB]
  cid = lax.axis_index("c")
  sid = lax.axis_index("s")
  wid = sid * NC + cid

  pltpu.sync_copy(src3d.at[wid], src_v)
  pltpu.sync_copy(dst3d.at[wid], dst_v)
  pltpu.sync_copy(z_hbm.at[pl.ds(sid * RPT, RPT)],
                  acc.at[pl.ds(sid * RPT, RPT)])
  plsc.subcore_barrier()

  for j in range(LA):
    pltpu.async_copy(g_hbm.at[src_v.at[j]], rows[j], gsems[j])

  def step(c0, carry):
    for b in range(NB):
      c = c0 * NB + b
      f = c + LA
      bf = (b + LA) % NB

      @pl.when(jnp.logical_and(f < CH, f >= NB))
      def _():
        pltpu.make_async_copy(rows[bf], acc.at[dst_v.at[f - NB]],
                              ssems[bf]).wait()

      @pl.when(f < CH)
      def _():
        pltpu.async_copy(g_hbm.at[src_v.at[f]], rows[bf], gsems[bf])

      pltpu.make_async_copy(g_hbm.at[src_v.at[c]], rows[b], gsems[b]).wait()
      pltpu.async_copy(rows[b], acc.at[dst_v.at[c]], ssems[b], add=True)
    return carry

  lax.fori_loop(0, CH // NB, step, 0)
  for b in range(NB):
    cl = CH - NB + b
    pltpu.make_async_copy(rows[b], acc.at[dst_v.at[cl]], ssems[b]).wait()
  plsc.subcore_barrier()
  pltpu.sync_copy(acc.at[pl.ds(sid * RPT, RPT)],
                  out_hbm.at[cid, pl.ds(sid * RPT, RPT)])


@functools.cache
def _make_agg(d, ch, nrows):
  rpt = nrows // NS
  mesh = plsc.VectorSubcoreMesh(core_axis_name="c", subcore_axis_name="s",
                                num_cores=NC, num_subcores=NS)
  return pl.kernel(
      functools.partial(_agg_body, ch, rpt),
      out_type=jax.ShapeDtypeStruct((NC, nrows, d), jnp.float32),
      mesh=mesh,
      compiler_params=pltpu.CompilerParams(use_tc_tiling_on_sc=False),
      scratch_types=[
          pltpu.VMEM_SHARED((nrows, d), jnp.float32),
          pltpu.VMEM((ch, C), jnp.int32),
          pltpu.VMEM((ch, C), jnp.int32),
          *[pltpu.VMEM((C, d), jnp.float32) for _ in range(NB)],
          *[pltpu.SemaphoreType.DMA for _ in range(2 * NB)],
      ],
  )


def _tc_head(n, dp_ref, x_ref, w_ref, g_ref, dinv_ref):
  deg = dp_ref[0, :n, 0:1] + dp_ref[1, :n, 0:1] + 1.0
  dinv = lax.rsqrt(jnp.maximum(deg, 1.0))
  g_ref[...] = dinv * jnp.dot(x_ref[...], w_ref[...],
                              preferred_element_type=jnp.float32)
  dinv_ref[...] = dinv


def _tc_mid(n, ap_ref, g_ref, dinv_ref, b_ref, w_ref, gout_ref):
  dinv = dinv_ref[...]
  t = dinv * (ap_ref[0, :n, :] + ap_ref[1, :n, :] + g_ref[...]) + b_ref[...]
  t = jnp.maximum(t, 0.0)
  gout_ref[...] = dinv * jnp.dot(t, w_ref[...],
                                 preferred_element_type=jnp.float32)


def _tc_tail(n, do, ap_ref, g_ref, dinv_ref, b_ref, out_ref):
  t = (dinv_ref[...] * (ap_ref[0, :n, :] + ap_ref[1, :n, :] + g_ref[...])
       + b_ref[...])
  out_ref[...] = t[:, :do]


def kernel(x, edge_index, W1, b1, W2, b2, W3, b3):
  n, d_in = x.shape
  e = edge_index.shape[1]
  dh = W1.shape[1]
  do = W3.shape[1]
  d3 = 8

  ch = -(-e // (NW * C))
  ch = -(-ch // NB) * NB
  pade = NW * ch * C
  nrows = -(-(n + 1) // (NS * 8)) * (NS * 8)

  src = edge_index[0]
  dst = edge_index[1]
  npad = pade - e
  srcp = jnp.concatenate([src, jnp.zeros((npad,), src.dtype)])
  dstp = jnp.concatenate([dst, jnp.full((npad,), n, dst.dtype)])
  src3d = srcp.reshape(NW, ch, C)
  dst3d = dstp.reshape(NW, ch, C)

  z64 = jnp.zeros((nrows, dh), jnp.float32)
  z8 = jnp.zeros((nrows, d3), jnp.float32)
  ones8 = jnp.ones((n, d3), jnp.float32)
  W3p = jnp.concatenate([W3, jnp.zeros((dh, d3 - do), W3.dtype)], axis=1)
  b3p = jnp.concatenate([b3, jnp.zeros((d3 - do,), b3.dtype)])

  agg64 = _make_agg(dh, ch, nrows)
  agg8 = _make_agg(d3, ch, nrows)

  degp = agg8(ones8, src3d, dst3d, z8)

  g1, dinv = pl.pallas_call(
      functools.partial(_tc_head, n),
      out_shape=(jax.ShapeDtypeStruct((n, dh), jnp.float32),
                 jax.ShapeDtypeStruct((n, 1), jnp.float32)),
  )(degp, x, W1)

  a1 = agg64(g1, src3d, dst3d, z64)
  g2 = pl.pallas_call(
      functools.partial(_tc_mid, n),
      out_shape=jax.ShapeDtypeStruct((n, dh), jnp.float32),
  )(a1, g1, dinv, b1.reshape(1, dh), W2)

  a2 = agg64(g2, src3d, dst3d, z64)
  g3 = pl.pallas_call(
      functools.partial(_tc_mid, n),
      out_shape=jax.ShapeDtypeStruct((n, d3), jnp.float32),
  )(a2, g2, dinv, b2.reshape(1, dh), W3p)

  a3 = agg8(g3, src3d, dst3d, z8)
  dx = pl.pallas_call(
      functools.partial(_tc_tail, n, do),
      out_shape=jax.ShapeDtypeStruct((n, do), jnp.float32),
  )(a3, g3, dinv, b3p.reshape(1, d3))

  return dx


# speedup vs baseline: 37.4098x; 1.8324x over previous
import functools

import jax
import jax.numpy as jnp
from jax import lax
from jax.experimental import pallas as pl
from jax.experimental.pallas import tpu as pltpu
from jax.experimental.pallas import tpu_sc as plsc

NC = 2
NS = 16
NW = NC * NS
C = 128
NB = 3
LA = 2


def _agg_body(CH, RPT, NT, g_hbm, src3d, dst3d, z_hbm, out_hbm, acc, tbl,
              src_v, dst_v, *bufs):
  rows = bufs[:NB]
  gsems = bufs[NB:2 * NB]
  ssems = bufs[2 * NB:3 * NB]
  cid = lax.axis_index("c")
  sid = lax.axis_index("s")
  wid = sid * NC + cid

  pltpu.sync_copy(src3d.at[wid], src_v)
  pltpu.sync_copy(dst3d.at[wid], dst_v)
  pltpu.sync_copy(z_hbm.at[pl.ds(sid * RPT, RPT)],
                  acc.at[pl.ds(sid * RPT, RPT)])
  last = NT - (NS - 1) * RPT

  @pl.when(sid < NS - 1)
  def _():
    pltpu.sync_copy(g_hbm.at[pl.ds(sid * RPT, RPT)],
                    tbl.at[pl.ds(sid * RPT, RPT)])

  @pl.when(sid == NS - 1)
  def _():
    pltpu.sync_copy(g_hbm.at[pl.ds((NS - 1) * RPT, last)],
                    tbl.at[pl.ds((NS - 1) * RPT, last)])

  plsc.subcore_barrier()

  for j in range(LA):
    pltpu.async_copy(tbl.at[src_v.at[j]], rows[j], gsems[j])

  def step(c0, carry):
    for b in range(NB):
      c = c0 * NB + b
      f = c + LA
      bf = (b + LA) % NB

      @pl.when(jnp.logical_and(f < CH, f >= NB))
      def _():
        pltpu.make_async_copy(rows[bf], acc.at[dst_v.at[f - NB]],
                              ssems[bf]).wait()

      @pl.when(f < CH)
      def _():
        pltpu.async_copy(tbl.at[src_v.at[f]], rows[bf], gsems[bf])

      pltpu.make_async_copy(tbl.at[src_v.at[c]], rows[b], gsems[b]).wait()
      pltpu.async_copy(rows[b], acc.at[dst_v.at[c]], ssems[b], add=True)
    return carry

  lax.fori_loop(0, CH // NB, step, 0)
  for b in range(NB):
    cl = CH - NB + b
    pltpu.make_async_copy(rows[b], acc.at[dst_v.at[cl]], ssems[b]).wait()
  plsc.subcore_barrier()
  pltpu.sync_copy(acc.at[pl.ds(sid * RPT, RPT)],
                  out_hbm.at[cid, pl.ds(sid * RPT, RPT)])


@functools.cache
def _make_agg(d, ch, nrows, nt):
  rpt = nrows // NS
  mesh = plsc.VectorSubcoreMesh(core_axis_name="c", subcore_axis_name="s",
                                num_cores=NC, num_subcores=NS)
  return pl.kernel(
      functools.partial(_agg_body, ch, rpt, nt),
      out_type=jax.ShapeDtypeStruct((NC, nrows, d), jnp.float32),
      mesh=mesh,
      compiler_params=pltpu.CompilerParams(use_tc_tiling_on_sc=False),
      scratch_types=[
          pltpu.VMEM_SHARED((nrows, d), jnp.float32),
          pltpu.VMEM_SHARED((nrows, d), jnp.float32),
          pltpu.VMEM((ch, C), jnp.int32),
          pltpu.VMEM((ch, C), jnp.int32),
          *[pltpu.VMEM((C, d), jnp.float32) for _ in range(NB)],
          *[pltpu.SemaphoreType.DMA for _ in range(2 * NB)],
      ],
  )


def _tc_head(n, dp_ref, x_ref, w_ref, g_ref, dinv_ref):
  deg = dp_ref[0, :n, 0:1] + dp_ref[1, :n, 0:1] + 1.0
  dinv = lax.rsqrt(jnp.maximum(deg, 1.0))
  g_ref[...] = dinv * jnp.dot(x_ref[...], w_ref[...],
                              preferred_element_type=jnp.float32)
  dinv_ref[...] = dinv


def _tc_mid(n, ap_ref, g_ref, dinv_ref, b_ref, w_ref, gout_ref):
  dinv = dinv_ref[...]
  t = dinv * (ap_ref[0, :n, :] + ap_ref[1, :n, :] + g_ref[...]) + b_ref[...]
  t = jnp.maximum(t, 0.0)
  gout_ref[...] = dinv * jnp.dot(t, w_ref[...],
                                 preferred_element_type=jnp.float32)


def _tc_tail(n, do, ap_ref, g_ref, dinv_ref, b_ref, out_ref):
  t = (dinv_ref[...] * (ap_ref[0, :n, :] + ap_ref[1, :n, :] + g_ref[...])
       + b_ref[...])
  out_ref[...] = t[:, :do]


def kernel(x, edge_index, W1, b1, W2, b2, W3, b3):
  n, d_in = x.shape
  e = edge_index.shape[1]
  dh = W1.shape[1]
  do = W3.shape[1]
  d3 = 8

  ch = -(-e // (NW * C))
  ch = -(-ch // NB) * NB
  pade = NW * ch * C
  nrows = -(-(n + 1) // (NS * 8)) * (NS * 8)

  src = edge_index[0]
  dst = edge_index[1]
  npad = pade - e
  srcp = jnp.concatenate([src, jnp.zeros((npad,), src.dtype)])
  dstp = jnp.concatenate([dst, jnp.full((npad,), n, dst.dtype)])
  src3d = srcp.reshape(NW, ch, C)
  dst3d = dstp.reshape(NW, ch, C)

  z64 = jnp.zeros((nrows, dh), jnp.float32)
  z8 = jnp.zeros((nrows, d3), jnp.float32)
  ones8 = jnp.ones((n, d3), jnp.float32)
  W3p = jnp.concatenate([W3, jnp.zeros((dh, d3 - do), W3.dtype)], axis=1)
  b3p = jnp.concatenate([b3, jnp.zeros((d3 - do,), b3.dtype)])

  agg64 = _make_agg(dh, ch, nrows, n)
  agg8 = _make_agg(d3, ch, nrows, n)

  degp = agg8(ones8, src3d, dst3d, z8)

  g1, dinv = pl.pallas_call(
      functools.partial(_tc_head, n),
      out_shape=(jax.ShapeDtypeStruct((n, dh), jnp.float32),
                 jax.ShapeDtypeStruct((n, 1), jnp.float32)),
  )(degp, x, W1)

  a1 = agg64(g1, src3d, dst3d, z64)
  g2 = pl.pallas_call(
      functools.partial(_tc_mid, n),
      out_shape=jax.ShapeDtypeStruct((n, dh), jnp.float32),
  )(a1, g1, dinv, b1.reshape(1, dh), W2)

  a2 = agg64(g2, src3d, dst3d, z64)
  g3 = pl.pallas_call(
      functools.partial(_tc_mid, n),
      out_shape=jax.ShapeDtypeStruct((n, d3), jnp.float32),
  )(a2, g2, dinv, b2.reshape(1, dh), W3p)

  a3 = agg8(g3, src3d, dst3d, z8)
  dx = pl.pallas_call(
      functools.partial(_tc_tail, n, do),
      out_shape=jax.ShapeDtypeStruct((n, do), jnp.float32),
  )(a3, g3, dinv, b3p.reshape(1, d3))

  return dx


# speedup vs baseline: 37.8063x; 1.0106x over previous
import functools

import jax
import jax.numpy as jnp
from jax import lax
from jax.experimental import pallas as pl
from jax.experimental.pallas import tpu as pltpu
from jax.experimental.pallas import tpu_sc as plsc

NC = 2
NS = 16
NW = NC * NS
C = 128
NB = 3
LA = 2


def _agg_body(Q, REM, RPT, NT, g_hbm, src3, dst3, z_hbm, out_hbm, acc, tbl,
              src_v, dst_v, *bufs):
  rows = bufs[:NB]
  gsems = bufs[NB:2 * NB]
  ssems = bufs[2 * NB:3 * NB]
  cid = lax.axis_index("c")
  sid = lax.axis_index("s")
  wid = sid * NC + cid
  base = wid * Q + jnp.minimum(wid, REM)

  pltpu.sync_copy(src3.at[pl.ds(base, Q)], src_v.at[pl.ds(0, Q)])
  pltpu.sync_copy(dst3.at[pl.ds(base, Q)], dst_v.at[pl.ds(0, Q)])

  @pl.when(wid < REM)
  def _():
    pltpu.sync_copy(src3.at[pl.ds(base + Q, 1)], src_v.at[pl.ds(Q, 1)])
    pltpu.sync_copy(dst3.at[pl.ds(base + Q, 1)], dst_v.at[pl.ds(Q, 1)])

  pltpu.sync_copy(z_hbm.at[pl.ds(sid * RPT, RPT)],
                  acc.at[pl.ds(sid * RPT, RPT)])
  last = NT - (NS - 1) * RPT

  @pl.when(sid < NS - 1)
  def _():
    pltpu.sync_copy(g_hbm.at[pl.ds(sid * RPT, RPT)],
                    tbl.at[pl.ds(sid * RPT, RPT)])

  @pl.when(sid == NS - 1)
  def _():
    pltpu.sync_copy(g_hbm.at[pl.ds((NS - 1) * RPT, last)],
                    tbl.at[pl.ds((NS - 1) * RPT, last)])

  plsc.subcore_barrier()

  for j in range(LA):
    pltpu.async_copy(tbl.at[src_v.at[j]], rows[j], gsems[j])

  def step(c0, carry):
    for b in range(NB):
      c = c0 * NB + b
      f = c + LA
      bf = (b + LA) % NB

      @pl.when(jnp.logical_and(f < Q, f >= NB))
      def _():
        pltpu.make_async_copy(rows[bf], acc.at[dst_v.at[f - NB]],
                              ssems[bf]).wait()

      @pl.when(f < Q)
      def _():
        pltpu.async_copy(tbl.at[src_v.at[f]], rows[bf], gsems[bf])

      pltpu.make_async_copy(tbl.at[src_v.at[c]], rows[b], gsems[b]).wait()
      pltpu.async_copy(rows[b], acc.at[dst_v.at[c]], ssems[b], add=True)
    return carry

  lax.fori_loop(0, Q // NB, step, 0)
  for b in range(NB):
    cl = Q - NB + b
    pltpu.make_async_copy(rows[b], acc.at[dst_v.at[cl]], ssems[b]).wait()

  @pl.when(wid < REM)
  def _():
    pltpu.async_copy(tbl.at[src_v.at[Q]], rows[0], gsems[0]).wait()
    pltpu.sync_copy(rows[0], acc.at[dst_v.at[Q]], add=True)

  plsc.subcore_barrier()
  pltpu.sync_copy(acc.at[pl.ds(sid * RPT, RPT)],
                  out_hbm.at[cid, pl.ds(sid * RPT, RPT)])


@functools.cache
def _make_agg(d, q, rem, nrows, nt):
  rpt = nrows // NS
  mesh = plsc.VectorSubcoreMesh(core_axis_name="c", subcore_axis_name="s",
                                num_cores=NC, num_subcores=NS)
  return pl.kernel(
      functools.partial(_agg_body, q, rem, rpt, nt),
      out_type=jax.ShapeDtypeStruct((NC, nrows, d), jnp.float32),
      mesh=mesh,
      compiler_params=pltpu.CompilerParams(use_tc_tiling_on_sc=False),
      scratch_types=[
          pltpu.VMEM_SHARED((nrows, d), jnp.float32),
          pltpu.VMEM_SHARED((nrows, d), jnp.float32),
          pltpu.VMEM((q + 1, C), jnp.int32),
          pltpu.VMEM((q + 1, C), jnp.int32),
          *[pltpu.VMEM((C, d), jnp.float32) for _ in range(NB)],
          *[pltpu.SemaphoreType.DMA for _ in range(2 * NB)],
      ],
  )


def _tc_mm(x_ref, w_ref, p_ref):
  p_ref[...] = jnp.dot(x_ref[...], w_ref[...],
                       preferred_element_type=jnp.float32)


def _tc_scale(n, dp_ref, p_ref, g_ref, dinv_ref):
  deg = dp_ref[0, :n, 0:1] + dp_ref[1, :n, 0:1] + 1.0
  dinv = lax.rsqrt(jnp.maximum(deg, 1.0))
  g_ref[...] = dinv * p_ref[...]
  dinv_ref[...] = dinv


def _tc_mid(n, ap_ref, g_ref, dinv_ref, b_ref, w_ref, gout_ref):
  dinv = dinv_ref[...]
  t = dinv * (ap_ref[0, :n, :] + ap_ref[1, :n, :] + g_ref[...]) + b_ref[...]
  t = jnp.maximum(t, 0.0)
  gout_ref[...] = dinv * jnp.dot(t, w_ref[...],
                                 preferred_element_type=jnp.float32)


def _tc_tail(n, do, ap_ref, g_ref, dinv_ref, b_ref, out_ref):
  t = (dinv_ref[...] * (ap_ref[0, :n, :] + ap_ref[1, :n, :] + g_ref[...])
       + b_ref[...])
  out_ref[...] = t[:, :do]


def kernel(x, edge_index, W1, b1, W2, b2, W3, b3):
  n, d_in = x.shape
  e = edge_index.shape[1]
  dh = W1.shape[1]
  do = W3.shape[1]
  d3 = 8

  if e % C == 0:
    ei = edge_index
  else:
    tail = C - e % C
    ei = jnp.concatenate(
        [edge_index,
         jnp.stack([jnp.zeros((tail,), edge_index.dtype),
                    jnp.full((tail,), n, edge_index.dtype)])], axis=1)
  tchunks = ei.shape[1] // C
  q = tchunks // NW
  q = (q // NB) * NB
  rem = tchunks - q * NW
  assert 0 <= rem <= NW, (tchunks, q, rem)
  ei3 = ei.reshape(2, tchunks, C)
  src3 = ei3[0]
  dst3 = ei3[1]

  nrows = -(-(n + 1) // (NS * 8)) * (NS * 8)

  z64 = jnp.zeros((nrows, dh), jnp.float32)
  z8 = jnp.zeros((nrows, d3), jnp.float32)
  ones8 = jnp.ones((n, d3), jnp.float32)
  W3p = jnp.concatenate([W3, jnp.zeros((dh, d3 - do), W3.dtype)], axis=1)
  b3p = jnp.concatenate([b3, jnp.zeros((d3 - do,), b3.dtype)])

  agg64 = _make_agg(dh, q, rem, nrows, n)
  agg8 = _make_agg(d3, q, rem, nrows, n)

  degp = agg8(ones8, src3, dst3, z8)
  p1 = pl.pallas_call(
      _tc_mm, out_shape=jax.ShapeDtypeStruct((n, dh), jnp.float32),
  )(x, W1)

  g1, dinv = pl.pallas_call(
      functools.partial(_tc_scale, n),
      out_shape=(jax.ShapeDtypeStruct((n, dh), jnp.float32),
                 jax.ShapeDtypeStruct((n, 1), jnp.float32)),
  )(degp, p1)

  a1 = agg64(g1, src3, dst3, z64)
  g2 = pl.pallas_call(
      functools.partial(_tc_mid, n),
      out_shape=jax.ShapeDtypeStruct((n, dh), jnp.float32),
  )(a1, g1, dinv, b1.reshape(1, dh), W2)

  a2 = agg64(g2, src3, dst3, z64)
  g3 = pl.pallas_call(
      functools.partial(_tc_mid, n),
      out_shape=jax.ShapeDtypeStruct((n, d3), jnp.float32),
  )(a2, g2, dinv, b2.reshape(1, dh), W3p)

  a3 = agg8(g3, src3, dst3, z8)
  dx = pl.pallas_call(
      functools.partial(_tc_tail, n, do),
      out_shape=jax.ShapeDtypeStruct((n, do), jnp.float32),
  )(a3, g3, dinv, b3p.reshape(1, d3))

  return dx


# speedup vs baseline: 42.0762x; 1.1129x over previous
import functools

import jax
import jax.numpy as jnp
from jax import lax
from jax.experimental import pallas as pl
from jax.experimental.pallas import tpu as pltpu
from jax.experimental.pallas import tpu_sc as plsc

NC = 2
NS = 16
NW = NC * NS
C = 128
NB = 3
LA = 2


def _agg_body(Q, REM, RPT, NT, D, PACK, g_hbm, ei3, z_hbm, out_hbm, acc, tbl,
              src_v, dst_v, *bufs):
  rows = bufs[:NB]
  gsems = bufs[NB:2 * NB]
  ssems = bufs[2 * NB:3 * NB]
  cid = lax.axis_index("c")
  sid = lax.axis_index("s")
  wid = sid * NC + cid
  base = wid * Q + jnp.minimum(wid, REM)

  pltpu.sync_copy(ei3.at[0, pl.ds(base, Q)], src_v.at[pl.ds(0, Q)])
  pltpu.sync_copy(ei3.at[1, pl.ds(base, Q)], dst_v.at[pl.ds(0, Q)])

  @pl.when(wid < REM)
  def _():
    pltpu.sync_copy(ei3.at[0, pl.ds(base + Q, 1)], src_v.at[pl.ds(Q, 1)])
    pltpu.sync_copy(ei3.at[1, pl.ds(base + Q, 1)], dst_v.at[pl.ds(Q, 1)])

  pltpu.sync_copy(z_hbm.at[pl.ds(sid * RPT, RPT)],
                  acc.at[pl.ds(sid * RPT, RPT)])
  last = NT - (NS - 1) * RPT

  @pl.when(sid < NS - 1)
  def _():
    pltpu.sync_copy(g_hbm.at[pl.ds(sid * RPT, RPT)],
                    tbl.at[pl.ds(sid * RPT, RPT)])

  @pl.when(sid == NS - 1)
  def _():
    pltpu.sync_copy(g_hbm.at[pl.ds((NS - 1) * RPT, last)],
                    tbl.at[pl.ds((NS - 1) * RPT, last)])

  plsc.subcore_barrier()

  for j in range(LA):
    pltpu.async_copy(tbl.at[src_v.at[j]], rows[j], gsems[j])

  def step(c0, carry):
    for b in range(NB):
      c = c0 * NB + b
      f = c + LA
      bf = (b + LA) % NB

      @pl.when(jnp.logical_and(f < Q, f >= NB))
      def _():
        pltpu.make_async_copy(rows[bf], acc.at[dst_v.at[f - NB]],
                              ssems[bf]).wait()

      @pl.when(f < Q)
      def _():
        pltpu.async_copy(tbl.at[src_v.at[f]], rows[bf], gsems[bf])

      pltpu.make_async_copy(tbl.at[src_v.at[c]], rows[b], gsems[b]).wait()
      pltpu.async_copy(rows[b], acc.at[dst_v.at[c]], ssems[b], add=True)
    return carry

  lax.fori_loop(0, Q // NB, step, 0)
  for b in range(NB):
    cl = Q - NB + b
    pltpu.make_async_copy(rows[b], acc.at[dst_v.at[cl]], ssems[b]).wait()

  @pl.when(wid < REM)
  def _():
    pltpu.async_copy(tbl.at[src_v.at[Q]], rows[0], gsems[0]).wait()
    pltpu.sync_copy(rows[0], acc.at[dst_v.at[Q]], add=True)

  plsc.subcore_barrier()
  if PACK:
    pltpu.sync_copy(acc.at[pl.ds(sid * RPT, RPT)],
                    out_hbm.at[pl.ds(sid * RPT, RPT), pl.ds(cid * D, D)])
  else:
    pltpu.sync_copy(acc.at[pl.ds(sid * RPT, RPT)],
                    out_hbm.at[cid, pl.ds(sid * RPT, RPT)])


@functools.cache
def _make_agg(d, q, rem, nrows, nt, pack):
  rpt = nrows // NS
  mesh = plsc.VectorSubcoreMesh(core_axis_name="c", subcore_axis_name="s",
                                num_cores=NC, num_subcores=NS)
  oshape = (nrows, NC * d) if pack else (NC, nrows, d)
  return pl.kernel(
      functools.partial(_agg_body, q, rem, rpt, nt, d, pack),
      out_type=jax.ShapeDtypeStruct(oshape, jnp.float32),
      mesh=mesh,
      compiler_params=pltpu.CompilerParams(use_tc_tiling_on_sc=False),
      scratch_types=[
          pltpu.VMEM_SHARED((nrows, d), jnp.float32),
          pltpu.VMEM_SHARED((nrows, d), jnp.float32),
          pltpu.VMEM((q + 1, C), jnp.int32),
          pltpu.VMEM((q + 1, C), jnp.int32),
          *[pltpu.VMEM((C, d), jnp.float32) for _ in range(NB)],
          *[pltpu.SemaphoreType.DMA for _ in range(2 * NB)],
      ],
  )


def _tc_mm(x_ref, w_ref, p_ref):
  p_ref[...] = jnp.dot(x_ref[...], w_ref[...],
                       preferred_element_type=jnp.float32)


def _tc_scale(n, dp_ref, p_ref, g_ref, dinv_ref):
  deg = dp_ref[0, :n, 0:1] + dp_ref[1, :n, 0:1] + 1.0
  dinv = lax.rsqrt(jnp.maximum(deg, 1.0))
  g_ref[...] = dinv * p_ref[...]
  dinv_ref[...] = dinv


def _tc_mid(n, dh, ap_ref, g_ref, dinv_ref, b_ref, w_ref, gout_ref):
  dinv = dinv_ref[...]
  t = (dinv * (ap_ref[:n, :dh] + ap_ref[:n, dh:] + g_ref[...])
       + b_ref[...])
  t = jnp.maximum(t, 0.0)
  gout_ref[...] = dinv * jnp.dot(t, w_ref[...],
                                 preferred_element_type=jnp.float32)


def _tc_tail(n, do, ap_ref, g_ref, dinv_ref, b_ref, out_ref):
  t = (dinv_ref[...] * (ap_ref[0, :n, :] + ap_ref[1, :n, :] + g_ref[...])
       + b_ref[...])
  out_ref[...] = t[:, :do]


def kernel(x, edge_index, W1, b1, W2, b2, W3, b3):
  n, d_in = x.shape
  e = edge_index.shape[1]
  dh = W1.shape[1]
  do = W3.shape[1]
  d3 = 8

  if e % C == 0:
    ei = edge_index
  else:
    tail = C - e % C
    ei = jnp.concatenate(
        [edge_index,
         jnp.stack([jnp.zeros((tail,), edge_index.dtype),
                    jnp.full((tail,), n, edge_index.dtype)])], axis=1)
  tchunks = ei.shape[1] // C
  q = tchunks // NW
  q = (q // NB) * NB
  rem = tchunks - q * NW
  assert 0 <= rem <= NW, (tchunks, q, rem)
  ei3 = ei.reshape(2, tchunks, C)

  nrows = -(-(n + 1) // (NS * 8)) * (NS * 8)

  z64 = jnp.zeros((nrows, dh), jnp.float32)
  z8 = jnp.zeros((nrows, d3), jnp.float32)
  ones8 = jnp.ones((n, d3), jnp.float32)
  W3p = jnp.concatenate([W3, jnp.zeros((dh, d3 - do), W3.dtype)], axis=1)
  b3p = jnp.concatenate([b3, jnp.zeros((d3 - do,), b3.dtype)])

  agg64 = _make_agg(dh, q, rem, nrows, n, True)
  agg8 = _make_agg(d3, q, rem, nrows, n, False)

  degp = agg8(ones8, ei3, z8)
  p1 = pl.pallas_call(
      _tc_mm, out_shape=jax.ShapeDtypeStruct((n, dh), jnp.float32),
  )(x, W1)

  g1, dinv = pl.pallas_call(
      functools.partial(_tc_scale, n),
      out_shape=(jax.ShapeDtypeStruct((n, dh), jnp.float32),
                 jax.ShapeDtypeStruct((n, 1), jnp.float32)),
  )(degp, p1)

  a1 = agg64(g1, ei3, z64)
  g2 = pl.pallas_call(
      functools.partial(_tc_mid, n, dh),
      out_shape=jax.ShapeDtypeStruct((n, dh), jnp.float32),
  )(a1, g1, dinv, b1.reshape(1, dh), W2)

  a2 = agg64(g2, ei3, z64)
  g3 = pl.pallas_call(
      functools.partial(_tc_mid, n, dh),
      out_shape=jax.ShapeDtypeStruct((n, d3), jnp.float32),
  )(a2, g2, dinv, b2.reshape(1, dh), W3p)

  a3 = agg8(g3, ei3, z8)
  dx = pl.pallas_call(
      functools.partial(_tc_tail, n, do),
      out_shape=jax.ShapeDtypeStruct((n, do), jnp.float32),
  )(a3, g3, dinv, b3p.reshape(1, d3))

  return dx


# speedup vs baseline: 50.4362x; 1.1987x over previous
import functools

import jax
import jax.numpy as jnp
from jax import lax
from jax.experimental import pallas as pl
from jax.experimental.pallas import tpu as pltpu
from jax.experimental.pallas import tpu_sc as plsc

NC = 2
NS = 16
NW = NC * NS
C = 128
NB = 3
LA = 2
LANES = 128


def _agg_body(Q0, Q1, REM, RPT, NT, D, MODE, g_hbm, ei3, z_hbm, out_hbm,
              acc, tbl, src_v, dst_v, *bufs):
  rows = bufs[:NB]
  gsems = bufs[NB:2 * NB]
  ssems = bufs[2 * NB:3 * NB]
  ones_v = bufs[3 * NB] if MODE == "const" else None
  cid = lax.axis_index("c")
  sid = lax.axis_index("s")
  wid = sid * NC + cid
  qmax = max(Q0, Q1)
  base = jnp.where(cid == 0, sid * Q0, NS * Q0 + sid * Q1)
  ebase = NS * (Q0 + Q1)

  @pl.when(cid == 0)
  def _():
    pltpu.sync_copy(ei3.at[0, pl.ds(base, Q0)], src_v.at[pl.ds(0, Q0)])
    pltpu.sync_copy(ei3.at[1, pl.ds(base, Q0)], dst_v.at[pl.ds(0, Q0)])

  @pl.when(cid == 1)
  def _():
    pltpu.sync_copy(ei3.at[0, pl.ds(base, Q1)], src_v.at[pl.ds(0, Q1)])
    pltpu.sync_copy(ei3.at[1, pl.ds(base, Q1)], dst_v.at[pl.ds(0, Q1)])

  @pl.when(wid < REM)
  def _():
    pltpu.sync_copy(ei3.at[0, pl.ds(ebase + wid, 1)],
                    src_v.at[pl.ds(qmax, 1)])
    pltpu.sync_copy(ei3.at[1, pl.ds(ebase + wid, 1)],
                    dst_v.at[pl.ds(qmax, 1)])

  pltpu.sync_copy(z_hbm.at[pl.ds(sid * RPT, RPT)],
                  acc.at[pl.ds(sid * RPT, RPT)])

  if MODE == "const":
    pltpu.sync_copy(g_hbm, ones_v)
  else:
    stage = (cid == 1) if MODE == "hybrid" else (cid >= 0)
    last = NT - (NS - 1) * RPT

    @pl.when(jnp.logical_and(stage, sid < NS - 1))
    def _():
      pltpu.sync_copy(g_hbm.at[pl.ds(sid * RPT, RPT)],
                      tbl.at[pl.ds(sid * RPT, RPT)])

    @pl.when(jnp.logical_and(stage, sid == NS - 1))
    def _():
      pltpu.sync_copy(g_hbm.at[pl.ds((NS - 1) * RPT, last)],
                      tbl.at[pl.ds((NS - 1) * RPT, last)])

  plsc.subcore_barrier()

  if MODE == "const":
    def cstep(c0, carry):
      for b in range(NB):
        c = c0 * NB + b

        @pl.when(c >= NB)
        def _():
          pltpu.make_async_copy(ones_v, acc.at[dst_v.at[c - NB]],
                                ssems[b]).wait()

        pltpu.async_copy(ones_v, acc.at[dst_v.at[c]], ssems[b], add=True)
      return carry

    lax.fori_loop(0, Q0 // NB, cstep, 0)
    for b in range(NB):
      pltpu.make_async_copy(ones_v, acc.at[dst_v.at[Q0 - NB + b]],
                            ssems[b]).wait()

    @pl.when(wid < REM)
    def _():
      pltpu.sync_copy(ones_v, acc.at[dst_v.at[qmax]], add=True)

  else:
    def pipeline(src_tbl, q):
      for j in range(LA):
        pltpu.async_copy(src_tbl.at[src_v.at[j]], rows[j], gsems[j])

      def step(c0, carry):
        for b in range(NB):
          c = c0 * NB + b
          f = c + LA
          bf = (b + LA) % NB

          @pl.when(jnp.logical_and(f < q, f >= NB))
          def _():
            pltpu.make_async_copy(rows[bf], acc.at[dst_v.at[f - NB]],
                                  ssems[bf]).wait()

          @pl.when(f < q)
          def _():
            pltpu.async_copy(src_tbl.at[src_v.at[f]], rows[bf], gsems[bf])

          pltpu.make_async_copy(src_tbl.at[src_v.at[c]], rows[b],
                                gsems[b]).wait()
          pltpu.async_copy(rows[b], acc.at[dst_v.at[c]], ssems[b],
                           add=True)
        return carry

      lax.fori_loop(0, q // NB, step, 0)
      for b in range(NB):
        pltpu.make_async_copy(rows[b], acc.at[dst_v.at[q - NB + b]],
                              ssems[b]).wait()

      @pl.when(wid < REM)
      def _():
        pltpu.async_copy(src_tbl.at[src_v.at[qmax]], rows[0],
                         gsems[0]).wait()
        pltpu.sync_copy(rows[0], acc.at[dst_v.at[qmax]], add=True)

    if MODE == "hybrid":
      @pl.when(cid == 0)
      def _():
        pipeline(g_hbm, Q0)

      @pl.when(cid == 1)
      def _():
        pipeline(tbl, Q1)
    else:
      pipeline(tbl, Q0)

  plsc.subcore_barrier()
  pltpu.sync_copy(acc.at[pl.ds(sid * RPT, RPT)],
                  out_hbm.at[pl.ds(sid * RPT, RPT), pl.ds(cid * D, D)])


@functools.cache
def _make_agg(d, q0, q1, rem, nrows, nt, mode):
  rpt = nrows // NS
  qmax = max(q0, q1)
  mesh = plsc.VectorSubcoreMesh(core_axis_name="c", subcore_axis_name="s",
                                num_cores=NC, num_subcores=NS)
  scratch = [
      pltpu.VMEM_SHARED((nrows, d), jnp.float32),
      pltpu.VMEM_SHARED((nt, d), jnp.float32),
      pltpu.VMEM((qmax + 1, C), jnp.int32),
      pltpu.VMEM((qmax + 1, C), jnp.int32),
      *[pltpu.VMEM((C, d), jnp.float32) for _ in range(NB)],
      *[pltpu.SemaphoreType.DMA for _ in range(2 * NB)],
  ]
  if mode == "const":
    scratch.append(pltpu.VMEM((C, d), jnp.float32))
  return pl.kernel(
      functools.partial(_agg_body, q0, q1, rem, rpt, nt, d, mode),
      out_type=jax.ShapeDtypeStruct((nrows, LANES), jnp.float32),
      mesh=mesh,
      compiler_params=pltpu.CompilerParams(use_tc_tiling_on_sc=False),
      scratch_types=scratch,
  )


def _tc_mm(x_ref, w_ref, p_ref):
  p_ref[...] = jnp.dot(x_ref[...], w_ref[...],
                       preferred_element_type=jnp.float32)


def _tc_scale(n, d3, dp_ref, p_ref, g_ref, dinv_ref):
  deg = dp_ref[:n, 0:1] + dp_ref[:n, d3:d3 + 1] + 1.0
  dinv = lax.rsqrt(jnp.maximum(deg, 1.0))
  g_ref[...] = dinv * p_ref[...]
  dinv_ref[...] = dinv


def _tc_mid(n, dh, ap_ref, g_ref, dinv_ref, b_ref, w_ref, gout_ref):
  dinv = dinv_ref[...]
  t = (dinv * (ap_ref[:n, :dh] + ap_ref[:n, dh:2 * dh] + g_ref[...])
       + b_ref[...])
  t = jnp.maximum(t, 0.0)
  gout_ref[...] = dinv * jnp.dot(t, w_ref[...],
                                 preferred_element_type=jnp.float32)


def _tc_tail(n, d3, do, ap_ref, g_ref, dinv_ref, b_ref, out_ref):
  t = (dinv_ref[...] * (ap_ref[:n, :d3] + ap_ref[:n, d3:2 * d3]
                        + g_ref[...]) + b_ref[...])
  out_ref[...] = t[:, :do]


def _splits(tchunks, ratio):
  s = tchunks // NS
  q0 = int(round(s * ratio / (1.0 + ratio) / NB)) * NB
  q0 = max(NB, min(q0, s - NB))
  q1 = ((s - q0) // NB) * NB
  rem = tchunks - NS * (q0 + q1)
  assert 0 <= rem <= NW and q0 >= NB and q1 >= NB, (tchunks, q0, q1, rem)
  return q0, q1, rem


def kernel(x, edge_index, W1, b1, W2, b2, W3, b3):
  n, d_in = x.shape
  e = edge_index.shape[1]
  dh = W1.shape[1]
  do = W3.shape[1]
  d3 = 8

  if e % C == 0:
    ei = edge_index
  else:
    tail = C - e % C
    ei = jnp.concatenate(
        [edge_index,
         jnp.stack([jnp.zeros((tail,), edge_index.dtype),
                    jnp.full((tail,), n, edge_index.dtype)])], axis=1)
  tchunks = ei.shape[1] // C
  ei3 = ei.reshape(2, tchunks, C)

  q0h, q1h, remh = _splits(tchunks, 1.55)
  q0s, q1s, rems = _splits(tchunks, 1.0)

  nrows = -(-(n + 1) // (NS * 8)) * (NS * 8)

  z64 = jnp.zeros((nrows, dh), jnp.float32)
  z8 = jnp.zeros((nrows, d3), jnp.float32)
  ones8 = jnp.ones((C, d3), jnp.float32)
  W3p = jnp.concatenate([W3, jnp.zeros((dh, d3 - do), W3.dtype)], axis=1)
  b3p = jnp.concatenate([b3, jnp.zeros((d3 - do,), b3.dtype)])

  agg64 = _make_agg(dh, q0h, q1h, remh, nrows, n, "hybrid")
  agg8 = _make_agg(d3, q0s, q1s, rems, nrows, n, "spmem")
  deg8 = _make_agg(d3, q0s, q1s, rems, nrows, n, "const")

  degp = deg8(ones8, ei3, z8)
  p1 = pl.pallas_call(
      _tc_mm, out_shape=jax.ShapeDtypeStruct((n, dh), jnp.float32),
  )(x, W1)

  g1, dinv = pl.pallas_call(
      functools.partial(_tc_scale, n, d3),
      out_shape=(jax.ShapeDtypeStruct((n, dh), jnp.float32),
                 jax.ShapeDtypeStruct((n, 1), jnp.float32)),
  )(degp, p1)

  a1 = agg64(g1, ei3, z64)
  g2 = pl.pallas_call(
      functools.partial(_tc_mid, n, dh),
      out_shape=jax.ShapeDtypeStruct((n, dh), jnp.float32),
  )(a1, g1, dinv, b1.reshape(1, dh), W2)

  a2 = agg64(g2, ei3, z64)
  g3 = pl.pallas_call(
      functools.partial(_tc_mid, n, dh),
      out_shape=jax.ShapeDtypeStruct((n, d3), jnp.float32),
  )(a2, g2, dinv, b2.reshape(1, dh), W3p)

  a3 = agg8(g3, ei3, z8)
  dx = pl.pallas_call(
      functools.partial(_tc_tail, n, d3, do),
      out_shape=jax.ShapeDtypeStruct((n, do), jnp.float32),
  )(a3, g3, dinv, b3p.reshape(1, d3))

  return dx
